# baseline (speedup 1.0000x reference)
import functools

import jax
import jax.numpy as jnp
from jax import lax
from jax.experimental import pallas as pl
from jax.experimental.pallas import tpu as pltpu
from jax.experimental.pallas import tpu_sc as plsc

N = 10000
E = 320000
D_IN = 128
H = 32
G = 10

NC = 2
NS = 16
NT = NC * NS
PT = E // NT
K = 400
NCH = PT // K
NBUF = 5
SLOTS = 2 * NBUF
NPAD = 10240
RPT = NPAD // NS
RQ = 128
NQ = RPT // RQ
DW = 32

def _zero_rows(zbuf, rows, width):
    def body(r, _):
        for c in range(width // 16):
            zbuf[r, pl.ds(c * 16, 16)] = jnp.zeros((16,), jnp.float32)
        return 0
    lax.fori_loop(0, rows, body, 0)


@functools.cache
def _make_deg_sc():
    mesh = plsc.VectorSubcoreMesh(core_axis_name="c", subcore_axis_name="s")
    return functools.partial(
        pl.kernel,
        mesh=mesh,
        out_type=jax.ShapeDtypeStruct((NC, NPAD, DW), jnp.float32),
        scratch_types=[
            pltpu.VMEM((NCH, K), jnp.int32),
            pltpu.VMEM((K, DW), jnp.float32),
            pltpu.VMEM((RQ, DW), jnp.float32),
            pltpu.VMEM_SHARED((NPAD, DW), jnp.float32),
            pltpu.SemaphoreType.DMA,
        ],
        compiler_params=pltpu.CompilerParams(use_tc_tiling_on_sc=False),
    )(_deg_sc_body)


def _deg_sc_body(ei_hbm, out_hbm, dst_v, ones_v, zbuf, acc_sh, sem):
    cid = lax.axis_index("c")
    sid = lax.axis_index("s")
    tile = cid * NS + sid
    base = tile * PT

    for j in range(NCH):
        pltpu.async_copy(ei_hbm.at[1, pl.ds(base + j * K, K)],
                         dst_v.at[j], sem)

    _zero_rows(zbuf, RQ, DW)

    def fill_ones(r, _):
        for c in range(DW // 16):
            ones_v[r, pl.ds(c * 16, 16)] = jnp.ones((16,), jnp.float32)
        return 0
    lax.fori_loop(0, K, fill_ones, 0)

    for q in range(NQ):
        pltpu.sync_copy(zbuf, acc_sh.at[pl.ds(sid * RPT + q * RQ, RQ)])

    for j in range(NCH):
        pltpu.make_async_copy(ei_hbm.at[1, pl.ds(0, K)], dst_v.at[j],
                              sem).wait()
    plsc.subcore_barrier()

    def body(j, _):
        pltpu.async_copy(ones_v, acc_sh.at[dst_v.at[j]], sem, add=True)
        return 0
    lax.fori_loop(0, NCH, body, 0)
    for j in range(NCH):
        pltpu.make_async_copy(out_hbm.at[0, pl.ds(0, K)], ones_v,
                              sem).wait()
    plsc.subcore_barrier()

    for q in range(NQ):
        row0 = sid * RPT + q * RQ
        pltpu.sync_copy(acc_sh.at[pl.ds(row0, RQ)], zbuf)
        pltpu.sync_copy(zbuf, out_hbm.at[cid, pl.ds(row0, RQ)])


@functools.cache
def _make_agg_sc():
    mesh = plsc.VectorSubcoreMesh(core_axis_name="c", subcore_axis_name="s")
    return functools.partial(
        pl.kernel,
        mesh=mesh,
        out_type=jax.ShapeDtypeStruct((NC, NPAD, H), jnp.float32),
        scratch_types=[
            pltpu.VMEM((PT,), jnp.int32),
            pltpu.VMEM((NCH, K), jnp.int32),
            pltpu.VMEM((NBUF, K, H), jnp.float32),
            pltpu.VMEM((RQ, H), jnp.float32),
            pltpu.VMEM_SHARED((NPAD, H), jnp.float32),
        ] + [pltpu.SemaphoreType.DMA] * (2 * SLOTS),
        compiler_params=pltpu.CompilerParams(use_tc_tiling_on_sc=False),
    )(_agg_sc_body)


def _agg_sc_body(hs_hbm, ei_hbm, out_hbm, src_v, dst_v, rows_v,
                 zbuf, acc_sh, *sems):
    cid = lax.axis_index("c")
    sid = lax.axis_index("s")
    tile = cid * NS + sid
    base = tile * PT
    gsems = sems[:SLOTS]
    ssems = sems[SLOTS:]

    for j in range(NCH):
        pltpu.async_copy(ei_hbm.at[1, pl.ds(base + j * K, K)],
                         dst_v.at[j], ssems[0])
    pltpu.sync_copy(ei_hbm.at[0, pl.ds(base, PT)], src_v)

    _zero_rows(zbuf, RQ, H)
    for q in range(NQ):
        pltpu.sync_copy(zbuf, acc_sh.at[pl.ds(sid * RPT + q * RQ, RQ)])

    for j in range(NCH):
        pltpu.make_async_copy(ei_hbm.at[1, pl.ds(0, K)], dst_v.at[j],
                              ssems[0]).wait()
    plsc.subcore_barrier()

    def gfire(j, s):
        pltpu.async_copy(hs_hbm.at[src_v.at[pl.ds(j * K, K)]],
                         rows_v.at[s], gsems[s])

    def gdrain(s):
        pltpu.make_async_copy(hs_hbm.at[pl.ds(0, K)], rows_v.at[s],
                              gsems[s]).wait()

    def sfire(j, s):
        pltpu.async_copy(rows_v.at[s], acc_sh.at[dst_v.at[j]], ssems[s],
                         add=True)

    def sdrain(s):
        pltpu.make_async_copy(hs_hbm.at[pl.ds(0, K)], rows_v.at[s],
                              ssems[s]).wait()

    def chunk(j, s, refill):
        gdrain(s)
        pltpu.sync_copy(rows_v.at[s], acc_sh.at[dst_v.at[j]], add=True)
        if refill:
            gfire(j + NBUF, s)

    for b in range(NBUF):
        gfire(b, b)

    def sgroup(t, _):
        j0 = t * NBUF
        for i in range(NBUF):
            chunk(j0 + i, i, refill=True)
        return 0
    lax.fori_loop(0, NCH // NBUF - 1, sgroup, 0)

    j0 = NCH - NBUF
    for i in range(NBUF):
        chunk(j0 + i, i, refill=False)
    plsc.subcore_barrier()

    for q in range(NQ):
        row0 = sid * RPT + q * RQ
        pltpu.sync_copy(acc_sh.at[pl.ds(row0, RQ)], zbuf)
        pltpu.sync_copy(zbuf, out_hbm.at[cid, pl.ds(row0, RQ)])


HROWS = N * H // 128
AROWS = NPAD * H // 128
GROWS = HROWS // G


def _dinv_pk(degp_ref):
    return lax.rsqrt(degp_ref[0] + degp_ref[1] + 1.0)


def _mm0_body(degp_ref, x_ref, w0e_ref, o_ref):
    x3 = x_ref[...].reshape(HROWS, 4, D_IN)
    h = sum(jnp.dot(x3[:, a, :], w0e_ref[a],
                    preferred_element_type=jnp.float32) for a in range(4))
    o_ref[...] = h * _dinv_pk(degp_ref)[:HROWS]


def _mid_body(aggp_ref, hs0_ref, degp_ref, b0_ref, gamma_ref, beta_ref,
              rm_ref, rv_ref, w1bd_ref, o_ref):
    dinv = _dinv_pk(degp_ref)[:HROWS]
    t = (aggp_ref[0, :HROWS] + aggp_ref[1, :HROWS] + hs0_ref[...]) * dinv \
        + b0_ref[...]
    t = jnp.maximum(t, 0.0)
    t = (t - rm_ref[...]) * lax.rsqrt(rv_ref[...] + 1e-5) * gamma_ref[...] \
        + beta_ref[...]
    h1 = jnp.dot(t, w1bd_ref[...], preferred_element_type=jnp.float32)
    o_ref[...] = h1 * dinv


def _final_body(aggp_ref, hs1_ref, degp_ref, b1_ref, jpool_ref, o_ref):
    dinv = _dinv_pk(degp_ref)[:HROWS]
    t = (aggp_ref[0, :HROWS] + aggp_ref[1, :HROWS] + hs1_ref[...]) * dinv \
        + b1_ref[...]
    t = jnp.maximum(t, 0.0)
    s = jnp.sum(t.reshape(G, GROWS, 128), axis=1)
    o_ref[...] = jnp.dot(s, jpool_ref[...],
                         preferred_element_type=jnp.float32) / float(N // G)


def kernel(x, edge_index, nodeNumList, W0, b0, gamma, beta, run_mean,
           run_var, W1, b1):
    degp = _make_deg_sc()(edge_index)

    degp_pk = degp.reshape(NC, AROWS, 128)
    tile4 = lambda v: jnp.tile(v, 4).reshape(1, 128)
    w1bd = jax.scipy.linalg.block_diag(W1, W1, W1, W1)
    w0e = jnp.stack([
        jnp.pad(W0, ((0, 0), (32 * a, 96 - 32 * a))) for a in range(4)
    ])
    jpool = jnp.concatenate([jnp.eye(H, dtype=jnp.float32)] * 4, axis=0)

    hs0_pk = pl.pallas_call(
        _mm0_body,
        out_shape=jax.ShapeDtypeStruct((HROWS, 128), jnp.float32),
    )(degp_pk, x, w0e)

    aggp0_pk = _make_agg_sc()(hs0_pk.reshape(N, H), edge_index).reshape(
        NC, AROWS, 128)

    hs1_pk = pl.pallas_call(
        _mid_body,
        out_shape=jax.ShapeDtypeStruct((HROWS, 128), jnp.float32),
    )(aggp0_pk, hs0_pk, degp_pk, tile4(b0), tile4(gamma), tile4(beta),
      tile4(run_mean), tile4(run_var), w1bd)

    aggp1_pk = _make_agg_sc()(hs1_pk.reshape(N, H), edge_index).reshape(
        NC, AROWS, 128)

    out = pl.pallas_call(
        _final_body,
        out_shape=jax.ShapeDtypeStruct((G, H), jnp.float32),
    )(aggp1_pk, hs1_pk, degp_pk, tile4(b1), jpool)

    return out


# speedup vs baseline: 1.0185x; 1.0185x over previous
import functools

import jax
import jax.numpy as jnp
from jax import lax
from jax.experimental import pallas as pl
from jax.experimental.pallas import tpu as pltpu
from jax.experimental.pallas import tpu_sc as plsc

N = 10000
E = 320000
D_IN = 128
H = 32
G = 10

NC = 2
NS = 16
NT = NC * NS
PT = E // NT
K = 200
NCH = PT // K
NBUF = 10
SLOTS = NBUF
NPAD = 10240
RPT = NPAD // NS
RQ = 128
NQ = RPT // RQ
DW = 32

def _zero_rows(zbuf, rows, width):
    def body(r, _):
        for c in range(width // 16):
            zbuf[r, pl.ds(c * 16, 16)] = jnp.zeros((16,), jnp.float32)
        return 0
    lax.fori_loop(0, rows, body, 0)


@functools.cache
def _make_deg_sc():
    mesh = plsc.VectorSubcoreMesh(core_axis_name="c", subcore_axis_name="s")
    return functools.partial(
        pl.kernel,
        mesh=mesh,
        out_type=jax.ShapeDtypeStruct((NC, NPAD, DW), jnp.float32),
        scratch_types=[
            pltpu.VMEM((NCH, K), jnp.int32),
            pltpu.VMEM((K, DW), jnp.float32),
            pltpu.VMEM((RQ, DW), jnp.float32),
            pltpu.VMEM_SHARED((NPAD, DW), jnp.float32),
            pltpu.SemaphoreType.DMA,
        ],
        compiler_params=pltpu.CompilerParams(use_tc_tiling_on_sc=False),
    )(_deg_sc_body)


def _deg_sc_body(ei_hbm, out_hbm, dst_v, ones_v, zbuf, acc_sh, sem):
    cid = lax.axis_index("c")
    sid = lax.axis_index("s")
    tile = cid * NS + sid
    base = tile * PT

    for j in range(NCH):
        pltpu.async_copy(ei_hbm.at[1, pl.ds(base + j * K, K)],
                         dst_v.at[j], sem)

    _zero_rows(zbuf, RQ, DW)

    def fill_ones(r, _):
        for c in range(DW // 16):
            ones_v[r, pl.ds(c * 16, 16)] = jnp.ones((16,), jnp.float32)
        return 0
    lax.fori_loop(0, K, fill_ones, 0)

    for q in range(NQ):
        pltpu.sync_copy(zbuf, acc_sh.at[pl.ds(sid * RPT + q * RQ, RQ)])

    for j in range(NCH):
        pltpu.make_async_copy(ei_hbm.at[1, pl.ds(0, K)], dst_v.at[j],
                              sem).wait()
    plsc.subcore_barrier()

    def body(j, _):
        pltpu.async_copy(ones_v, acc_sh.at[dst_v.at[j]], sem, add=True)
        return 0
    lax.fori_loop(0, NCH, body, 0)
    for j in range(NCH):
        pltpu.make_async_copy(out_hbm.at[0, pl.ds(0, K)], ones_v,
                              sem).wait()
    plsc.subcore_barrier()

    for q in range(NQ):
        row0 = sid * RPT + q * RQ
        pltpu.sync_copy(acc_sh.at[pl.ds(row0, RQ)], zbuf)
        pltpu.sync_copy(zbuf, out_hbm.at[cid, pl.ds(row0, RQ)])


@functools.cache
def _make_agg_sc():
    mesh = plsc.VectorSubcoreMesh(core_axis_name="c", subcore_axis_name="s")
    return functools.partial(
        pl.kernel,
        mesh=mesh,
        out_type=jax.ShapeDtypeStruct((NC, NPAD, H), jnp.float32),
        scratch_types=[
            pltpu.VMEM((PT,), jnp.int32),
            pltpu.VMEM((NCH, K), jnp.int32),
            pltpu.VMEM((NBUF, K, H), jnp.float32),
            pltpu.VMEM((RQ, H), jnp.float32),
            pltpu.VMEM_SHARED((NPAD, H), jnp.float32),
        ] + [pltpu.SemaphoreType.DMA] * (2 * SLOTS),
        compiler_params=pltpu.CompilerParams(use_tc_tiling_on_sc=False),
    )(_agg_sc_body)


def _agg_sc_body(hs_hbm, ei_hbm, out_hbm, src_v, dst_v, rows_v,
                 zbuf, acc_sh, *sems):
    cid = lax.axis_index("c")
    sid = lax.axis_index("s")
    tile = cid * NS + sid
    base = tile * PT
    gsems = sems[:SLOTS]
    ssems = sems[SLOTS:]

    for j in range(NCH):
        pltpu.async_copy(ei_hbm.at[1, pl.ds(base + j * K, K)],
                         dst_v.at[j], ssems[0])
    pltpu.sync_copy(ei_hbm.at[0, pl.ds(base, PT)], src_v)

    _zero_rows(zbuf, RQ, H)
    for q in range(NQ):
        pltpu.sync_copy(zbuf, acc_sh.at[pl.ds(sid * RPT + q * RQ, RQ)])

    for j in range(NCH):
        pltpu.make_async_copy(ei_hbm.at[1, pl.ds(0, K)], dst_v.at[j],
                              ssems[0]).wait()
    plsc.subcore_barrier()

    def gfire(j, s):
        pltpu.async_copy(hs_hbm.at[src_v.at[pl.ds(j * K, K)]],
                         rows_v.at[s], gsems[s])

    def gdrain(s):
        pltpu.make_async_copy(hs_hbm.at[pl.ds(0, K)], rows_v.at[s],
                              gsems[s]).wait()

    def sfire(j, s):
        pltpu.async_copy(rows_v.at[s], acc_sh.at[dst_v.at[j]], ssems[s],
                         add=True)

    def sdrain(s):
        pltpu.make_async_copy(hs_hbm.at[pl.ds(0, K)], rows_v.at[s],
                              ssems[s]).wait()

    def chunk(j, s, refill):
        gdrain(s)
        pltpu.sync_copy(rows_v.at[s], acc_sh.at[dst_v.at[j]], add=True)
        if refill:
            gfire(j + NBUF, s)

    for b in range(NBUF):
        gfire(b, b)

    def sgroup(t, _):
        j0 = t * NBUF
        for i in range(NBUF):
            chunk(j0 + i, i, refill=True)
        return 0
    lax.fori_loop(0, NCH // NBUF - 1, sgroup, 0)

    j0 = NCH - NBUF
    for i in range(NBUF):
        chunk(j0 + i, i, refill=False)
    plsc.subcore_barrier()

    for q in range(NQ):
        row0 = sid * RPT + q * RQ
        pltpu.sync_copy(acc_sh.at[pl.ds(row0, RQ)], zbuf)
        pltpu.sync_copy(zbuf, out_hbm.at[cid, pl.ds(row0, RQ)])


HROWS = N * H // 128
AROWS = NPAD * H // 128
GROWS = HROWS // G


def _dinv_pk(degp_ref):
    return lax.rsqrt(degp_ref[0] + degp_ref[1] + 1.0)


def _mm0_body(degp_ref, x_ref, w0e_ref, o_ref):
    x3 = x_ref[...].reshape(HROWS, 4, D_IN)
    h = sum(jnp.dot(x3[:, a, :], w0e_ref[a],
                    preferred_element_type=jnp.float32) for a in range(4))
    o_ref[...] = h * _dinv_pk(degp_ref)[:HROWS]


def _mid_body(aggp_ref, hs0_ref, degp_ref, b0_ref, gamma_ref, beta_ref,
              rm_ref, rv_ref, w1bd_ref, o_ref):
    dinv = _dinv_pk(degp_ref)[:HROWS]
    t = (aggp_ref[0, :HROWS] + aggp_ref[1, :HROWS] + hs0_ref[...]) * dinv \
        + b0_ref[...]
    t = jnp.maximum(t, 0.0)
    t = (t - rm_ref[...]) * lax.rsqrt(rv_ref[...] + 1e-5) * gamma_ref[...] \
        + beta_ref[...]
    h1 = jnp.dot(t, w1bd_ref[...], preferred_element_type=jnp.float32)
    o_ref[...] = h1 * dinv


def _final_body(aggp_ref, hs1_ref, degp_ref, b1_ref, jpool_ref, o_ref):
    dinv = _dinv_pk(degp_ref)[:HROWS]
    t = (aggp_ref[0, :HROWS] + aggp_ref[1, :HROWS] + hs1_ref[...]) * dinv \
        + b1_ref[...]
    t = jnp.maximum(t, 0.0)
    s = jnp.sum(t.reshape(G, GROWS, 128), axis=1)
    o_ref[...] = jnp.dot(s, jpool_ref[...],
                         preferred_element_type=jnp.float32) / float(N // G)


def kernel(x, edge_index, nodeNumList, W0, b0, gamma, beta, run_mean,
           run_var, W1, b1):
    degp = _make_deg_sc()(edge_index)

    degp_pk = degp.reshape(NC, AROWS, 128)
    tile4 = lambda v: jnp.tile(v, 4).reshape(1, 128)
    w1bd = jax.scipy.linalg.block_diag(W1, W1, W1, W1)
    w0e = jnp.stack([
        jnp.pad(W0, ((0, 0), (32 * a, 96 - 32 * a))) for a in range(4)
    ])
    jpool = jnp.concatenate([jnp.eye(H, dtype=jnp.float32)] * 4, axis=0)

    hs0_pk = pl.pallas_call(
        _mm0_body,
        out_shape=jax.ShapeDtypeStruct((HROWS, 128), jnp.float32),
    )(degp_pk, x, w0e)

    aggp0_pk = _make_agg_sc()(hs0_pk.reshape(N, H), edge_index).reshape(
        NC, AROWS, 128)

    hs1_pk = pl.pallas_call(
        _mid_body,
        out_shape=jax.ShapeDtypeStruct((HROWS, 128), jnp.float32),
    )(aggp0_pk, hs0_pk, degp_pk, tile4(b0), tile4(gamma), tile4(beta),
      tile4(run_mean), tile4(run_var), w1bd)

    aggp1_pk = _make_agg_sc()(hs1_pk.reshape(N, H), edge_index).reshape(
        NC, AROWS, 128)

    out = pl.pallas_call(
        _final_body,
        out_shape=jax.ShapeDtypeStruct((G, H), jnp.float32),
    )(aggp1_pk, hs1_pk, degp_pk, tile4(b1), jpool)

    return out


# speedup vs baseline: 1.0185x; 1.0000x over previous
import functools

import jax
import jax.numpy as jnp
from jax import lax
from jax.experimental import pallas as pl
from jax.experimental.pallas import tpu as pltpu
from jax.experimental.pallas import tpu_sc as plsc

N = 10000
E = 320000
D_IN = 128
H = 32
G = 10

NC = 2
NS = 16
NT = NC * NS
PT = E // NT
K = 200
NCH = PT // K
NBUF = 10
SLOTS = NBUF
NPAD = 10240
RPT = NPAD // NS
RQ = 128
NQ = RPT // RQ
DW = 32

def _zero_rows(zbuf, rows, width):
    def body(r, _):
        for c in range(width // 16):
            zbuf[r, pl.ds(c * 16, 16)] = jnp.zeros((16,), jnp.float32)
        return 0
    lax.fori_loop(0, rows, body, 0)


@functools.cache
def _make_deg_sc():
    mesh = plsc.VectorSubcoreMesh(core_axis_name="c", subcore_axis_name="s")
    return functools.partial(
        pl.kernel,
        mesh=mesh,
        out_type=jax.ShapeDtypeStruct((NC, NPAD, DW), jnp.float32),
        scratch_types=[
            pltpu.VMEM((NCH, K), jnp.int32),
            pltpu.VMEM((K, DW), jnp.float32),
            pltpu.VMEM((RQ, DW), jnp.float32),
            pltpu.VMEM_SHARED((NPAD, DW), jnp.float32),
            pltpu.SemaphoreType.DMA,
        ],
        compiler_params=pltpu.CompilerParams(use_tc_tiling_on_sc=False),
    )(_deg_sc_body)


def _deg_sc_body(ei_hbm, out_hbm, dst_v, ones_v, zbuf, acc_sh, sem):
    cid = lax.axis_index("c")
    sid = lax.axis_index("s")
    tile = cid * NS + sid
    base = tile * PT

    for j in range(NCH):
        pltpu.async_copy(ei_hbm.at[1, pl.ds(base + j * K, K)],
                         dst_v.at[j], sem)

    _zero_rows(zbuf, RQ, DW)

    def fill_ones(r, _):
        for c in range(DW // 16):
            ones_v[r, pl.ds(c * 16, 16)] = jnp.ones((16,), jnp.float32)
        return 0
    lax.fori_loop(0, K, fill_ones, 0)

    for q in range(NQ):
        pltpu.sync_copy(zbuf, acc_sh.at[pl.ds(sid * RPT + q * RQ, RQ)])

    for j in range(NCH):
        pltpu.make_async_copy(ei_hbm.at[1, pl.ds(0, K)], dst_v.at[j],
                              sem).wait()
    plsc.subcore_barrier()

    def body(j, _):
        pltpu.async_copy(ones_v, acc_sh.at[dst_v.at[j]], sem, add=True)
        return 0
    lax.fori_loop(0, NCH, body, 0)
    for j in range(NCH):
        pltpu.make_async_copy(out_hbm.at[0, pl.ds(0, K)], ones_v,
                              sem).wait()
    plsc.subcore_barrier()

    for q in range(NQ):
        row0 = sid * RPT + q * RQ
        pltpu.sync_copy(acc_sh.at[pl.ds(row0, RQ)], zbuf)
        pltpu.sync_copy(zbuf, out_hbm.at[cid, pl.ds(row0, RQ)])


@functools.cache
def _make_agg_sc():
    mesh = plsc.VectorSubcoreMesh(core_axis_name="c", subcore_axis_name="s")
    return functools.partial(
        pl.kernel,
        mesh=mesh,
        out_type=jax.ShapeDtypeStruct((NC, NPAD, H), jnp.float32),
        scratch_types=[
            pltpu.VMEM((PT,), jnp.int32),
            pltpu.VMEM((NCH, K), jnp.int32),
            pltpu.VMEM((NBUF, K, H), jnp.float32),
            pltpu.VMEM((RQ, H), jnp.float32),
            pltpu.VMEM_SHARED((NPAD, H), jnp.float32),
        ] + [pltpu.SemaphoreType.DMA] * (SLOTS + 1),
        compiler_params=pltpu.CompilerParams(use_tc_tiling_on_sc=False),
    )(_agg_sc_body)


def _agg_sc_body(hs_hbm, ei_hbm, out_hbm, src_v, dst_v, rows_v,
                 zbuf, acc_sh, *sems):
    cid = lax.axis_index("c")
    sid = lax.axis_index("s")
    tile = cid * NS + sid
    base = tile * PT
    gsems = sems[:SLOTS]
    dsem = sems[SLOTS]

    for j in range(NCH):
        pltpu.async_copy(ei_hbm.at[1, pl.ds(base + j * K, K)],
                         dst_v.at[j], dsem)
    pltpu.sync_copy(ei_hbm.at[0, pl.ds(base, PT)], src_v)

    _zero_rows(zbuf, RQ, H)
    for q in range(NQ):
        pltpu.sync_copy(zbuf, acc_sh.at[pl.ds(sid * RPT + q * RQ, RQ)])

    for j in range(NCH):
        pltpu.make_async_copy(ei_hbm.at[1, pl.ds(0, K)], dst_v.at[j],
                              dsem).wait()
    plsc.subcore_barrier()

    def gfire(j, s):
        pltpu.async_copy(hs_hbm.at[src_v.at[pl.ds(j * K, K)]],
                         rows_v.at[s], gsems[s])

    def gdrain(s):
        pltpu.make_async_copy(hs_hbm.at[pl.ds(0, K)], rows_v.at[s],
                              gsems[s]).wait()

    def chunk(j, s, refill):
        gdrain(s)
        pltpu.sync_copy(rows_v.at[s], acc_sh.at[dst_v.at[j]], add=True)
        if refill:
            gfire(j + NBUF, s)

    for b in range(NBUF):
        gfire(b, b)

    def sgroup(t, _):
        j0 = t * NBUF
        for i in range(NBUF):
            chunk(j0 + i, i, refill=True)
        return 0
    lax.fori_loop(0, NCH // NBUF - 1, sgroup, 0)

    j0 = NCH - NBUF
    for i in range(NBUF):
        chunk(j0 + i, i, refill=False)
    plsc.subcore_barrier()

    for q in range(NQ):
        row0 = sid * RPT + q * RQ
        pltpu.sync_copy(acc_sh.at[pl.ds(row0, RQ)], zbuf)
        pltpu.sync_copy(zbuf, out_hbm.at[cid, pl.ds(row0, RQ)])


HROWS = N * H // 128
AROWS = NPAD * H // 128
GROWS = HROWS // G


def _dinv_pk(degp_ref):
    return lax.rsqrt(degp_ref[0] + degp_ref[1] + 1.0)


def _mm0_body(degp_ref, x_ref, w0e_ref, o_ref):
    x3 = x_ref[...].reshape(HROWS, 4, D_IN)
    h = sum(jnp.dot(x3[:, a, :], w0e_ref[a],
                    preferred_element_type=jnp.float32) for a in range(4))
    o_ref[...] = h * _dinv_pk(degp_ref)[:HROWS]


def _mid_body(aggp_ref, hs0_ref, degp_ref, b0_ref, gamma_ref, beta_ref,
              rm_ref, rv_ref, w1bd_ref, o_ref):
    dinv = _dinv_pk(degp_ref)[:HROWS]
    t = (aggp_ref[0, :HROWS] + aggp_ref[1, :HROWS] + hs0_ref[...]) * dinv \
        + b0_ref[...]
    t = jnp.maximum(t, 0.0)
    t = (t - rm_ref[...]) * lax.rsqrt(rv_ref[...] + 1e-5) * gamma_ref[...] \
        + beta_ref[...]
    h1 = jnp.dot(t, w1bd_ref[...], preferred_element_type=jnp.float32)
    o_ref[...] = h1 * dinv


def _final_body(aggp_ref, hs1_ref, degp_ref, b1_ref, jpool_ref, o_ref):
    dinv = _dinv_pk(degp_ref)[:HROWS]
    t = (aggp_ref[0, :HROWS] + aggp_ref[1, :HROWS] + hs1_ref[...]) * dinv \
        + b1_ref[...]
    t = jnp.maximum(t, 0.0)
    s = jnp.sum(t.reshape(G, GROWS, 128), axis=1)
    o_ref[...] = jnp.dot(s, jpool_ref[...],
                         preferred_element_type=jnp.float32) / float(N // G)


def kernel(x, edge_index, nodeNumList, W0, b0, gamma, beta, run_mean,
           run_var, W1, b1):
    degp = _make_deg_sc()(edge_index)

    degp_pk = degp.reshape(NC, AROWS, 128)
    tile4 = lambda v: jnp.tile(v, 4).reshape(1, 128)
    w1bd = jax.scipy.linalg.block_diag(W1, W1, W1, W1)
    w0e = jnp.stack([
        jnp.pad(W0, ((0, 0), (32 * a, 96 - 32 * a))) for a in range(4)
    ])
    jpool = jnp.concatenate([jnp.eye(H, dtype=jnp.float32)] * 4, axis=0)

    hs0_pk = pl.pallas_call(
        _mm0_body,
        out_shape=jax.ShapeDtypeStruct((HROWS, 128), jnp.float32),
    )(degp_pk, x, w0e)

    aggp0_pk = _make_agg_sc()(hs0_pk.reshape(N, H), edge_index).reshape(
        NC, AROWS, 128)

    hs1_pk = pl.pallas_call(
        _mid_body,
        out_shape=jax.ShapeDtypeStruct((HROWS, 128), jnp.float32),
    )(aggp0_pk, hs0_pk, degp_pk, tile4(b0), tile4(gamma), tile4(beta),
      tile4(run_mean), tile4(run_var), w1bd)

    aggp1_pk = _make_agg_sc()(hs1_pk.reshape(N, H), edge_index).reshape(
        NC, AROWS, 128)

    out = pl.pallas_call(
        _final_body,
        out_shape=jax.ShapeDtypeStruct((G, H), jnp.float32),
    )(aggp1_pk, hs1_pk, degp_pk, tile4(b1), jpool)

    return out
